# baseline (speedup 1.0000x reference)
import jax
import jax.numpy as jnp
from jax import lax
from jax.experimental import pallas as pl
from jax.experimental.pallas import tpu as pltpu

TS_START = 26
N_TABLE = 126
N_REP = 20
N_TS = N_TABLE - TS_START
F_OUT = TS_START + N_TS * N_REP
DIM = 64
B_BLK = 4
DEPTH = 4


def _bias_kernel(table_ref, full_ref):
    full_ref[0:TS_START] = table_ref[0:TS_START]
    ts = table_ref[TS_START:N_TABLE]
    for r in range(N_REP):
        base = TS_START + r * N_TS
        full_ref[base:base + N_TS] = ts


def _stream_kernel(x_hbm, bias_ref, o_hbm, *bufs_and_sems):
    in_bufs = bufs_and_sems[0:DEPTH]
    out_bufs = bufs_and_sems[DEPTH:2 * DEPTH]
    in_sems = bufs_and_sems[2 * DEPTH]
    out_sems = bufs_and_sems[2 * DEPTH + 1]
    n_blocks = x_hbm.shape[0] // B_BLK

    def in_copy(i, d):
        return pltpu.make_async_copy(
            x_hbm.at[pl.ds(i * B_BLK, B_BLK)], in_bufs[d], in_sems.at[d])

    def out_copy(i, d):
        return pltpu.make_async_copy(
            out_bufs[d], o_hbm.at[pl.ds(i * B_BLK, B_BLK)], out_sems.at[d])

    for d in range(DEPTH):
        in_copy(d, d).start()

    def step(g, carry):
        for d in range(DEPTH):
            i = g * DEPTH + d
            in_copy(i, d).wait()

            @pl.when(i >= DEPTH)
            def _wait_prev_out():
                out_copy(i - DEPTH, d).wait()

            out_bufs[d][...] = in_bufs[d][...] + bias_ref[...][None, :, :]
            out_copy(i, d).start()

            @pl.when(i + DEPTH < n_blocks)
            def _start_next_in():
                in_copy(i + DEPTH, d).start()

        return carry

    lax.fori_loop(0, n_blocks // DEPTH, step, 0)
    for d in range(DEPTH):
        out_copy(n_blocks - DEPTH + d, d).wait()


def kernel(X, table):
    B = X.shape[0]
    full2d = pl.pallas_call(
        _bias_kernel,
        out_shape=jax.ShapeDtypeStruct((F_OUT, DIM), table.dtype),
    )(table)
    return pl.pallas_call(
        _stream_kernel,
        in_specs=[
            pl.BlockSpec(memory_space=pl.ANY),
            pl.BlockSpec(memory_space=pltpu.MemorySpace.VMEM),
        ],
        out_specs=pl.BlockSpec(memory_space=pl.ANY),
        out_shape=jax.ShapeDtypeStruct((B, F_OUT, DIM), X.dtype),
        scratch_shapes=(
            [pltpu.VMEM((B_BLK, F_OUT, DIM), X.dtype) for _ in range(2 * DEPTH)]
            + [pltpu.SemaphoreType.DMA((DEPTH,)),
               pltpu.SemaphoreType.DMA((DEPTH,))]
        ),
        compiler_params=pltpu.CompilerParams(
            vmem_limit_bytes=100 * 1024 * 1024,
        ),
    )(X, full2d)
